# baseline (speedup 1.0000x reference)
import jax
import jax.numpy as jnp
import numpy as np
from jax import lax
from jax.experimental import pallas as pl
from jax.experimental.pallas import tpu as pltpu
from jax.experimental.pallas import tpu_sc as plsc

VOCAB = 100000
D_MODEL = 128
SEQ_LEN = 2048
BATCH = 4

_NC = 2
_NS = 16
_NW = _NC * _NS
_SPAN = SEQ_LEN // _NW
_DV = D_MODEL // 16


def _build_pe() -> np.ndarray:
    pos = np.arange(SEQ_LEN)[:, None].astype(np.float64)
    i = np.arange(D_MODEL)[None, :].astype(np.float64)
    angle = pos / np.power(10000.0, (2.0 * (i // 2)) / D_MODEL)
    pe = np.zeros((SEQ_LEN, D_MODEL), dtype=np.float32)
    pe[:, 0::2] = np.sin(angle[:, 0::2])
    pe[:, 1::2] = np.cos(angle[:, 1::2])
    return pe


_PE = _build_pe()


def _embed_body(w_hbm, x1_hbm, x2_hbm, pe_hbm, out1_hbm, out2_hbm,
                idx1_v, idx2_v, pe_v, rows1_v, rows2_v, *sems):
    wid = lax.axis_index("s") * _NC + lax.axis_index("c")
    s0 = wid * _SPAN

    cp_pe = pltpu.async_copy(pe_hbm.at[pl.ds(s0, _SPAN)], pe_v, sems[0])
    idx_cps = []
    for b in range(BATCH):
        idx_cps.append(pltpu.async_copy(
            x1_hbm.at[b, pl.ds(s0, _SPAN)], idx1_v.at[b], sems[1]))
        idx_cps.append(pltpu.async_copy(
            x2_hbm.at[b, pl.ds(s0, _SPAN)], idx2_v.at[b], sems[2]))
    for cp in idx_cps:
        cp.wait()

    gathers = []
    for b in range(BATCH):
        for idx_v, rows_v in ((idx1_v, rows1_v), (idx2_v, rows2_v)):
            sem = sems[3 + len(gathers)]
            gathers.append(
                (pltpu.async_copy(w_hbm.at[idx_v.at[b]], rows_v.at[b], sem),
                 sem))

    cp_pe.wait()
    stores = []
    for b in range(BATCH):
        for k, (rows_v, out_hbm) in enumerate(((rows1_v, out1_hbm),
                                               (rows2_v, out2_hbm))):
            cp, sem = gathers[2 * b + k]
            cp.wait()

            @plsc.parallel_loop(0, _SPAN, step=1)
            def _(r):
                for d in range(_DV):
                    dsl = pl.ds(d * 16, 16)
                    plsc.addupdate(rows_v.at[b, r, dsl], pe_v[r, dsl])

            stores.append(pltpu.async_copy(
                rows_v.at[b], out_hbm.at[b, pl.ds(s0, _SPAN)], sem))
    for st in stores:
        st.wait()


_sc_embed = pl.kernel(
    _embed_body,
    out_type=(
        jax.ShapeDtypeStruct((BATCH, SEQ_LEN, D_MODEL), jnp.float32),
        jax.ShapeDtypeStruct((BATCH, SEQ_LEN, D_MODEL), jnp.float32),
    ),
    mesh=plsc.VectorSubcoreMesh(core_axis_name="c", subcore_axis_name="s"),
    scratch_types=[
        pltpu.VMEM((BATCH, _SPAN), jnp.int32),
        pltpu.VMEM((BATCH, _SPAN), jnp.int32),
        pltpu.VMEM((_SPAN, D_MODEL), jnp.float32),
        pltpu.VMEM((BATCH, _SPAN, D_MODEL), jnp.float32),
        pltpu.VMEM((BATCH, _SPAN, D_MODEL), jnp.float32),
    ] + [pltpu.SemaphoreType.DMA] * (3 + 2 * BATCH),
)


@jax.jit
def kernel(x, x_output, W):
    pe = jnp.asarray(_PE)
    return _sc_embed(W, x, x_output, pe)


# speedup vs baseline: 1.0262x; 1.0262x over previous
import jax
import jax.numpy as jnp
import numpy as np
from jax import lax
from jax.experimental import pallas as pl
from jax.experimental.pallas import tpu as pltpu
from jax.experimental.pallas import tpu_sc as plsc

VOCAB = 100000
D_MODEL = 128
SEQ_LEN = 2048
BATCH = 4

_NC = 2
_NS = 16
_NW = _NC * _NS
_SPAN = SEQ_LEN // _NW
_DV = D_MODEL // 16


def _build_pe() -> np.ndarray:
    pos = np.arange(SEQ_LEN)[:, None].astype(np.float64)
    i = np.arange(D_MODEL)[None, :].astype(np.float64)
    angle = pos / np.power(10000.0, (2.0 * (i // 2)) / D_MODEL)
    pe = np.zeros((SEQ_LEN, D_MODEL), dtype=np.float32)
    pe[:, 0::2] = np.sin(angle[:, 0::2])
    pe[:, 1::2] = np.cos(angle[:, 1::2])
    return pe


_PE = _build_pe()


def _embed_body(w_hbm, x1_hbm, x2_hbm, pe_hbm, out1_hbm, out2_hbm,
                idx1_v, idx2_v, pe_v, rows1_v, rows2_v, *sems):
    wid = lax.axis_index("s") * _NC + lax.axis_index("c")
    s0 = wid * _SPAN

    idx_cps = []
    for b in range(BATCH):
        idx_cps.append(pltpu.async_copy(
            x1_hbm.at[b, pl.ds(s0, _SPAN)], idx1_v.at[b], sems[1]))
        idx_cps.append(pltpu.async_copy(
            x2_hbm.at[b, pl.ds(s0, _SPAN)], idx2_v.at[b], sems[2]))
    cp_pe = pltpu.async_copy(pe_hbm.at[pl.ds(s0, _SPAN)], pe_v, sems[0])
    for cp in idx_cps:
        cp.wait()

    gathers = []
    for b in range(BATCH):
        for idx_v, rows_v in ((idx1_v, rows1_v), (idx2_v, rows2_v)):
            sem = sems[3 + len(gathers)]
            gathers.append(
                (pltpu.async_copy(w_hbm.at[idx_v.at[b]], rows_v.at[b], sem),
                 sem))

    cp_pe.wait()
    stores = []
    for b in range(BATCH):
        (cp1, sem1), (cp2, sem2) = gathers[2 * b], gathers[2 * b + 1]
        cp1.wait()
        cp2.wait()

        @plsc.parallel_loop(0, _SPAN, step=1)
        def _(r):
            for rr in range(1):
                for d in range(_DV):
                    dsl = pl.ds(d * 16, 16)
                    pv = pe_v[r + rr, dsl]
                    plsc.addupdate(rows1_v.at[b, r + rr, dsl], pv)
                    plsc.addupdate(rows2_v.at[b, r + rr, dsl], pv)

        stores.append(pltpu.async_copy(
            rows1_v.at[b], out1_hbm.at[b, pl.ds(s0, _SPAN)], sem1))
        stores.append(pltpu.async_copy(
            rows2_v.at[b], out2_hbm.at[b, pl.ds(s0, _SPAN)], sem2))
    for st in stores:
        st.wait()


_sc_embed = pl.kernel(
    _embed_body,
    out_type=(
        jax.ShapeDtypeStruct((BATCH, SEQ_LEN, D_MODEL), jnp.float32),
        jax.ShapeDtypeStruct((BATCH, SEQ_LEN, D_MODEL), jnp.float32),
    ),
    mesh=plsc.VectorSubcoreMesh(core_axis_name="c", subcore_axis_name="s"),
    scratch_types=[
        pltpu.VMEM((BATCH, _SPAN), jnp.int32),
        pltpu.VMEM((BATCH, _SPAN), jnp.int32),
        pltpu.VMEM((_SPAN, D_MODEL), jnp.float32),
        pltpu.VMEM((BATCH, _SPAN, D_MODEL), jnp.float32),
        pltpu.VMEM((BATCH, _SPAN, D_MODEL), jnp.float32),
    ] + [pltpu.SemaphoreType.DMA] * (3 + 2 * BATCH),
)


@jax.jit
def kernel(x, x_output, W):
    pe = jnp.asarray(_PE)
    return _sc_embed(W, x, x_output, pe)
